# baseline (speedup 1.0000x reference)
import functools

import jax
import jax.numpy as jnp
from jax import lax
from jax.experimental import pallas as pl
from jax.experimental.pallas import tpu as pltpu
from jax.experimental.pallas import tpu_sc as plsc

NCORE, NSUB, LANES = 2, 16, 16
NW = NCORE * NSUB


def _make_gather(B, NT, D, R):
    WPB = NW // B
    HALF = R // WPB
    CH = 128
    NCH = HALF // CH
    mesh = plsc.VectorSubcoreMesh(core_axis_name="c", subcore_axis_name="s",
                                  num_cores=NCORE, num_subcores=NSUB)

    @functools.partial(
        pl.kernel,
        mesh=mesh,
        out_type=(
            jax.ShapeDtypeStruct((B, R, D), jnp.float32),
            jax.ShapeDtypeStruct((B, R), jnp.float32),
        ),
        scratch_types=[
            pltpu.VMEM((R,), jnp.int32),
            pltpu.VMEM((NT,), jnp.float32),
            pltpu.VMEM((HALF,), jnp.float32),
            pltpu.VMEM((CH, D), jnp.float32),
            pltpu.VMEM((CH, D), jnp.float32),
            pltpu.SemaphoreType.DMA,
            pltpu.SemaphoreType.DMA,
            pltpu.SemaphoreType.DMA,
            pltpu.SemaphoreType.DMA,
        ],
        compiler_params=pltpu.CompilerParams(needs_layout_passes=False),
    )
    def gather_k(x_hbm, idx_hbm, sel_hbm, scale_hbm, idx_v, counts_v,
                 scale_v, buf0, buf1, gsem0, gsem1, osem0, osem1):
        wid = lax.axis_index("s") * NCORE + lax.axis_index("c")
        b = wid // WPB
        base = (wid % WPB) * HALF

        pltpu.sync_copy(idx_hbm.at[b], idx_v)

        bufs = (buf0, buf1)
        gsems = (gsem0, gsem1)
        osems = (osem0, osem1)

        def gath(g, p):
            return pltpu.async_copy(
                x_hbm.at[b].at[idx_v.at[pl.ds(base + g * CH, CH)]],
                bufs[p], gsems[p])

        gd = [gath(0, 0), None]
        od = [None, None]

        zeros16 = jnp.zeros((LANES,), jnp.float32)
        ones16 = jnp.ones((LANES,), jnp.float32)

        def zero_body(i, _):
            counts_v[pl.ds(i * LANES, LANES)] = zeros16
            return 0
        lax.fori_loop(0, NT // LANES, zero_body, 0, unroll=4)

        def hist_body(i, _):
            v = idx_v[pl.ds(i * LANES, LANES)]
            plsc.addupdate_scatter(counts_v, [v], ones16)
            return 0
        lax.fori_loop(0, R // LANES, hist_body, 0, unroll=4)

        def scale_body(i, _):
            iv = idx_v[pl.ds(base + i * LANES, LANES)]
            c = plsc.load_gather(counts_v, [iv])
            scale_v[pl.ds(i * LANES, LANES)] = 1.0 / c
            return 0
        lax.fori_loop(0, HALF // LANES, scale_body, 0, unroll=4)
        pltpu.sync_copy(scale_v, scale_hbm.at[b].at[pl.ds(base, HALF)])

        for g in range(NCH):
            p = g % 2
            gd[p].wait()
            if g + 1 < NCH:
                q = (g + 1) % 2
                if od[q] is not None:
                    od[q].wait()
                gd[q] = gath(g + 1, q)
            od[p] = pltpu.async_copy(
                bufs[p], sel_hbm.at[b].at[pl.ds(base + g * CH, CH)], osems[p])
        od[0].wait()
        od[1].wait()

    return gather_k


def _make_scatter(B, NT, D, R):
    PB = B // NCORE
    RT = NT // NSUB
    G = 64
    ZR = 64
    mesh = plsc.VectorSubcoreMesh(core_axis_name="c", subcore_axis_name="s",
                                  num_cores=NCORE, num_subcores=NSUB)

    @functools.partial(
        pl.kernel,
        mesh=mesh,
        out_type=jax.ShapeDtypeStruct((B, NT, D), jnp.float32),
        scratch_types=[
            pltpu.VMEM((RT, D), jnp.float32),
            pltpu.VMEM((3, G, D), jnp.float32),
            pltpu.VMEM((R,), jnp.int32),
            pltpu.VMEM((R + LANES,), jnp.int32),
            pltpu.VMEM((R + LANES,), jnp.int32),
            pltpu.SemaphoreType.DMA,
            pltpu.SemaphoreType.DMA,
            pltpu.SemaphoreType.DMA,
            pltpu.SemaphoreType.DMA,
            pltpu.SemaphoreType.DMA,
        ],
        compiler_params=pltpu.CompilerParams(needs_layout_passes=False),
    )
    def scatter_k(y_hbm, idx_hbm, out_hbm, acc_v, gbuf, idx_v,
                  mid_v, off_v, sem0, sem1, sem2, isem, osem):
        core = lax.axis_index("c")
        s = lax.axis_index("s")
        lo = s * RT
        zeros16 = jnp.zeros((LANES,), jnp.float32)
        zeros16i = jnp.zeros((LANES,), jnp.int32)
        sems = (sem0, sem1, sem2)

        def zm_body(i, _):
            mid_v[pl.ds(i * LANES, LANES)] = zeros16i
            return 0
        lax.fori_loop(0, (R + LANES) // LANES, zm_body, 0, unroll=4)

        def gstart(b, g, p, sp):
            pltpu.async_copy(
                y_hbm.at[b].at[mid_v.at[pl.ds(g * G, G)]], gbuf.at[p],
                sems[sp])

        def gwait(b, g, p, sp):
            pltpu.make_async_copy(
                y_hbm.at[b].at[mid_v.at[pl.ds(g * G, G)]], gbuf.at[p],
                sems[sp]).wait()

        pltpu.async_copy(idx_hbm.at[core * PB], idx_v, isem)

        def batch_body(i, _):
            b = core * PB + i

            with jax.named_scope("ph_idx"):
                pltpu.make_async_copy(idx_hbm.at[b], idx_v, isem).wait()

            def comp(j, n):
                v = idx_v[pl.ds(j * LANES, LANES)]
                m = (v >= lo) & (v < lo + RT)
                ids = lax.iota(jnp.int32, LANES) + j * LANES
                plsc.store_compressed(mid_v.at[pl.ds(n, LANES)], ids, mask=m)
                plsc.store_compressed(off_v.at[pl.ds(n, LANES)], v - lo, mask=m)
                return n + plsc.all_reduce_population_count(m)[0]
            with jax.named_scope("ph_comp"):
                n = lax.fori_loop(0, R // LANES, comp, 0, unroll=2)

            @pl.when(i + 1 < PB)
            def _():
                pltpu.async_copy(idx_hbm.at[b + 1], idx_v, isem)

            @pl.when(i > 0)
            def _():
                pltpu.make_async_copy(
                    acc_v, out_hbm.at[b].at[pl.ds(lo, RT)], osem).wait()

            def zacc(r, _):
                for c in range(D // LANES):
                    acc_v[r, pl.ds(c * LANES, LANES)] = zeros16
                return 0
            with jax.named_scope("ph_zacc"):
                lax.fori_loop(0, RT, zacc, 0, unroll=2)

            nch = (n + G - 1) // G

            for pp in range(3):
                @pl.when(nch > pp)
                def _(pp=pp):
                    gstart(b, pp, pp, pp)

            def chunk(g, _):
                p = g % 3

                with jax.named_scope("ph_gwait"):
                    for sp in range(3):
                        @pl.when(p == sp)
                        def _(sp=sp):
                            gwait(b, g, p, sp)

                nr = jnp.minimum(n - g * G, G)
                nfull = nr // LANES

                with jax.named_scope("ph_apply"):
                    def apply_full(q, _):
                        off16 = off_v[pl.ds(g * G + q * LANES, LANES)]
                        for j in range(LANES):
                            off = off16[j]
                            row = q * LANES + j
                            for c in range(D // LANES):
                                sl = pl.ds(c * LANES, LANES)
                                plsc.addupdate(acc_v.at[off, sl],
                                               gbuf[p, row, sl])
                        return 0
                    lax.fori_loop(0, nfull, apply_full, 0)

                    @pl.when(nfull * LANES < nr)
                    def _():
                        qq = nfull
                        off16 = off_v[pl.ds(g * G + qq * LANES, LANES)]
                        for j in range(LANES):
                            row = qq * LANES + j

                            @pl.when(row < nr)
                            def _():
                                off = off16[j]
                                for c in range(D // LANES):
                                    sl = pl.ds(c * LANES, LANES)
                                    plsc.addupdate(acc_v.at[off, sl],
                                                   gbuf[p, row, sl])

                for sp in range(3):
                    @pl.when((g + 3 < nch) & (p == sp))
                    def _(sp=sp):
                        gstart(b, g + 3, p, sp)
                return 0
            with jax.named_scope("ph_chunks"):
                lax.fori_loop(0, nch, chunk, 0)

            with jax.named_scope("ph_out"):
                pltpu.async_copy(acc_v, out_hbm.at[b].at[pl.ds(lo, RT)], osem)
            return 0

        lax.fori_loop(0, PB, batch_body, 0)
        pltpu.make_async_copy(
            acc_v, out_hbm.at[core * PB + PB - 1].at[pl.ds(lo, RT)],
            osem).wait()

    return scatter_k


def _make_mm(BK, NS, D, KB=8):
    def body(adj_ref, sel_ref, scale_ref, out_ref):
        for j in range(KB):
            a = adj_ref[j]
            v = sel_ref[j]
            sc = scale_ref[0, j, :]
            out_ref[j] = (
                jnp.dot(a, v, preferred_element_type=jnp.float32) * sc[:, None])

    return pl.pallas_call(
        body,
        grid=(BK // KB,),
        in_specs=[
            pl.BlockSpec((KB, NS, NS), lambda i: (i, 0, 0)),
            pl.BlockSpec((KB, NS, D), lambda i: (i, 0, 0)),
            pl.BlockSpec((1, KB, NS), lambda i: (i, 0, 0)),
        ],
        out_specs=pl.BlockSpec((KB, NS, D), lambda i: (i, 0, 0)),
        out_shape=jax.ShapeDtypeStruct((BK, NS, D), jnp.float32),
    )


def kernel(x, adj, batch_indices, indices):
    B, NT, D = x.shape
    _, K, NS, _ = adj.shape
    R = K * NS
    idx_flat = indices.reshape(B, R)

    KB = 8
    H = 2
    Bh = B // H
    gather_h = _make_gather(Bh, NT, D, R)
    mm_h = _make_mm(Bh * K, NS, D, KB)
    scatter_h = _make_scatter(Bh, NT, D, R)

    outs = []
    for h in range(H):
        sb = slice(h * Bh, (h + 1) * Bh)
        selh, scaleh = gather_h(x[sb], idx_flat[sb])
        yh = mm_h(
            adj[sb].reshape(Bh * K, NS, NS),
            selh.reshape(Bh * K, NS, D),
            scaleh.reshape(Bh * K // KB, KB, NS),
        )
        outs.append(scatter_h(yh.reshape(Bh, R, D), idx_flat[sb]))
    return jnp.concatenate(outs, axis=0)


# speedup vs baseline: 1.2726x; 1.2726x over previous
import functools

import jax
import jax.numpy as jnp
from jax import lax
from jax.experimental import pallas as pl
from jax.experimental.pallas import tpu as pltpu
from jax.experimental.pallas import tpu_sc as plsc

NCORE, NSUB, LANES = 2, 16, 16
NW = NCORE * NSUB


def _make_gather(B, NT, D, R):
    WPB = NW // B
    HALF = R // WPB
    CH = 128
    NCH = HALF // CH
    mesh = plsc.VectorSubcoreMesh(core_axis_name="c", subcore_axis_name="s",
                                  num_cores=NCORE, num_subcores=NSUB)

    @functools.partial(
        pl.kernel,
        mesh=mesh,
        out_type=(
            jax.ShapeDtypeStruct((B, R, D), jnp.float32),
            jax.ShapeDtypeStruct((B, R), jnp.float32),
        ),
        scratch_types=[
            pltpu.VMEM((R,), jnp.int32),
            pltpu.VMEM((NT,), jnp.float32),
            pltpu.VMEM((HALF,), jnp.float32),
            pltpu.VMEM((CH, D), jnp.float32),
            pltpu.VMEM((CH, D), jnp.float32),
            pltpu.SemaphoreType.DMA,
            pltpu.SemaphoreType.DMA,
            pltpu.SemaphoreType.DMA,
            pltpu.SemaphoreType.DMA,
        ],
        compiler_params=pltpu.CompilerParams(needs_layout_passes=False),
    )
    def gather_k(x_hbm, idx_hbm, sel_hbm, scale_hbm, idx_v, counts_v,
                 scale_v, buf0, buf1, gsem0, gsem1, osem0, osem1):
        wid = lax.axis_index("s") * NCORE + lax.axis_index("c")
        b = wid // WPB
        base = (wid % WPB) * HALF

        pltpu.sync_copy(idx_hbm.at[b], idx_v)

        bufs = (buf0, buf1)
        gsems = (gsem0, gsem1)
        osems = (osem0, osem1)

        def gath(g, p):
            return pltpu.async_copy(
                x_hbm.at[b].at[idx_v.at[pl.ds(base + g * CH, CH)]],
                bufs[p], gsems[p])

        gd = [gath(0, 0), None]
        od = [None, None]

        zeros16 = jnp.zeros((LANES,), jnp.float32)
        ones16 = jnp.ones((LANES,), jnp.float32)

        def zero_body(i, _):
            counts_v[pl.ds(i * LANES, LANES)] = zeros16
            return 0
        lax.fori_loop(0, NT // LANES, zero_body, 0, unroll=4)

        def hist_body(i, _):
            v = idx_v[pl.ds(i * LANES, LANES)]
            plsc.addupdate_scatter(counts_v, [v], ones16)
            return 0
        lax.fori_loop(0, R // LANES, hist_body, 0, unroll=4)

        def scale_body(i, _):
            iv = idx_v[pl.ds(base + i * LANES, LANES)]
            c = plsc.load_gather(counts_v, [iv])
            scale_v[pl.ds(i * LANES, LANES)] = 1.0 / c
            return 0
        lax.fori_loop(0, HALF // LANES, scale_body, 0, unroll=4)
        pltpu.sync_copy(scale_v, scale_hbm.at[b].at[pl.ds(base, HALF)])

        for g in range(NCH):
            p = g % 2
            gd[p].wait()
            if g + 1 < NCH:
                q = (g + 1) % 2
                if od[q] is not None:
                    od[q].wait()
                gd[q] = gath(g + 1, q)
            od[p] = pltpu.async_copy(
                bufs[p], sel_hbm.at[b].at[pl.ds(base + g * CH, CH)], osems[p])
        od[0].wait()
        od[1].wait()

    return gather_k


def _make_scatter(B, NT, D, R):
    PB = B // NCORE
    RT = NT // NSUB
    G = 64
    ZR = 64
    mesh = plsc.VectorSubcoreMesh(core_axis_name="c", subcore_axis_name="s",
                                  num_cores=NCORE, num_subcores=NSUB)

    @functools.partial(
        pl.kernel,
        mesh=mesh,
        out_type=jax.ShapeDtypeStruct((B, NT, D), jnp.float32),
        scratch_types=[
            pltpu.VMEM((RT, D), jnp.float32),
            pltpu.VMEM((3, G, D), jnp.float32),
            pltpu.VMEM((R,), jnp.int32),
            pltpu.VMEM((R + LANES,), jnp.int32),
            pltpu.VMEM((R + LANES,), jnp.int32),
            pltpu.SemaphoreType.DMA,
            pltpu.SemaphoreType.DMA,
            pltpu.SemaphoreType.DMA,
            pltpu.SemaphoreType.DMA,
            pltpu.SemaphoreType.DMA,
        ],
        compiler_params=pltpu.CompilerParams(needs_layout_passes=False),
    )
    def scatter_k(y_hbm, idx_hbm, out_hbm, acc_v, gbuf, idx_v,
                  mid_v, off_v, sem0, sem1, sem2, isem, osem):
        core = lax.axis_index("c")
        s = lax.axis_index("s")
        lo = s * RT
        zeros16 = jnp.zeros((LANES,), jnp.float32)
        zeros16i = jnp.zeros((LANES,), jnp.int32)
        sems = (sem0, sem1, sem2)

        def zm_body(i, _):
            mid_v[pl.ds(i * LANES, LANES)] = zeros16i
            return 0
        lax.fori_loop(0, (R + LANES) // LANES, zm_body, 0, unroll=4)

        def gstart(b, g, p, sp):
            pltpu.async_copy(
                y_hbm.at[b].at[mid_v.at[pl.ds(g * G, G)]], gbuf.at[p],
                sems[sp])

        def gwait(b, g, p, sp):
            pltpu.make_async_copy(
                y_hbm.at[b].at[mid_v.at[pl.ds(g * G, G)]], gbuf.at[p],
                sems[sp]).wait()

        pltpu.async_copy(idx_hbm.at[core * PB], idx_v, isem)

        def batch_body(i, _):
            b = core * PB + i

            with jax.named_scope("ph_idx"):
                pltpu.make_async_copy(idx_hbm.at[b], idx_v, isem).wait()

            def comp(j, n):
                v = idx_v[pl.ds(j * LANES, LANES)]
                m = (v >= lo) & (v < lo + RT)
                ids = lax.iota(jnp.int32, LANES) + j * LANES
                plsc.store_compressed(mid_v.at[pl.ds(n, LANES)], ids, mask=m)
                plsc.store_compressed(off_v.at[pl.ds(n, LANES)], v - lo, mask=m)
                return n + plsc.all_reduce_population_count(m)[0]
            with jax.named_scope("ph_comp"):
                n = lax.fori_loop(0, R // LANES, comp, 0, unroll=2)

            @pl.when(i + 1 < PB)
            def _():
                pltpu.async_copy(idx_hbm.at[b + 1], idx_v, isem)

            @pl.when(i > 0)
            def _():
                pltpu.make_async_copy(
                    acc_v, out_hbm.at[b].at[pl.ds(lo, RT)], osem).wait()

            def zacc(r, _):
                for c in range(D // LANES):
                    acc_v[r, pl.ds(c * LANES, LANES)] = zeros16
                return 0
            with jax.named_scope("ph_zacc"):
                lax.fori_loop(0, RT, zacc, 0, unroll=2)

            nch = (n + G - 1) // G

            for pp in range(3):
                @pl.when(nch > pp)
                def _(pp=pp):
                    gstart(b, pp, pp, pp)

            def chunk(g, _):
                p = g % 3

                with jax.named_scope("ph_gwait"):
                    for sp in range(3):
                        @pl.when(p == sp)
                        def _(sp=sp):
                            gwait(b, g, p, sp)

                nr = jnp.minimum(n - g * G, G)
                nfull = nr // LANES

                with jax.named_scope("ph_apply"):
                    def apply_full(q, _):
                        off16 = off_v[pl.ds(g * G + q * LANES, LANES)]
                        for j in range(0, LANES, 2):
                            offa = off16[j]
                            offb = off16[j + 1]
                            rowa = q * LANES + j
                            rowb = rowa + 1
                            for c in range(D // LANES):
                                sl = pl.ds(c * LANES, LANES)
                                plsc.addupdate(acc_v.at[offa, sl],
                                               gbuf[p, rowa, sl])
                                plsc.addupdate(acc_v.at[offb, sl],
                                               gbuf[p, rowb, sl])
                        return 0
                    lax.fori_loop(0, nfull, apply_full, 0)

                    @pl.when(nfull * LANES < nr)
                    def _():
                        qq = nfull
                        off16 = off_v[pl.ds(g * G + qq * LANES, LANES)]
                        for j in range(LANES):
                            row = qq * LANES + j

                            @pl.when(row < nr)
                            def _():
                                off = off16[j]
                                for c in range(D // LANES):
                                    sl = pl.ds(c * LANES, LANES)
                                    plsc.addupdate(acc_v.at[off, sl],
                                                   gbuf[p, row, sl])

                for sp in range(3):
                    @pl.when((g + 3 < nch) & (p == sp))
                    def _(sp=sp):
                        gstart(b, g + 3, p, sp)
                return 0
            with jax.named_scope("ph_chunks"):
                lax.fori_loop(0, nch, chunk, 0)

            with jax.named_scope("ph_out"):
                pltpu.async_copy(acc_v, out_hbm.at[b].at[pl.ds(lo, RT)], osem)
            return 0

        lax.fori_loop(0, PB, batch_body, 0)
        pltpu.make_async_copy(
            acc_v, out_hbm.at[core * PB + PB - 1].at[pl.ds(lo, RT)],
            osem).wait()

    return scatter_k


def _make_mm(BK, NS, D, KB=8):
    def body(adj_ref, sel_ref, scale_ref, out_ref):
        for j in range(KB):
            a = adj_ref[j]
            v = sel_ref[j]
            sc = scale_ref[0, j, :]
            out_ref[j] = (
                jnp.dot(a, v, preferred_element_type=jnp.float32) * sc[:, None])

    return pl.pallas_call(
        body,
        grid=(BK // KB,),
        in_specs=[
            pl.BlockSpec((KB, NS, NS), lambda i: (i, 0, 0)),
            pl.BlockSpec((KB, NS, D), lambda i: (i, 0, 0)),
            pl.BlockSpec((1, KB, NS), lambda i: (i, 0, 0)),
        ],
        out_specs=pl.BlockSpec((KB, NS, D), lambda i: (i, 0, 0)),
        out_shape=jax.ShapeDtypeStruct((BK, NS, D), jnp.float32),
    )


def kernel(x, adj, batch_indices, indices):
    B, NT, D = x.shape
    _, K, NS, _ = adj.shape
    R = K * NS
    idx_flat = indices.reshape(B, R)

    KB = 8
    sel, scale = _make_gather(B, NT, D, R)(x, idx_flat)
    y = _make_mm(B * K, NS, D, KB)(
        adj.reshape(B * K, NS, NS),
        sel.reshape(B * K, NS, D),
        scale.reshape(B * K // KB, KB, NS),
    )
    return _make_scatter(B, NT, D, R)(y.reshape(B, R, D), idx_flat)


# speedup vs baseline: 1.3444x; 1.0565x over previous
import functools

import jax
import jax.numpy as jnp
from jax import lax
from jax.experimental import pallas as pl
from jax.experimental.pallas import tpu as pltpu
from jax.experimental.pallas import tpu_sc as plsc

NCORE, NSUB, LANES = 2, 16, 16
NW = NCORE * NSUB


def _make_gather(B, NT, D, R):
    WPB = NW // B
    HALF = R // WPB
    CH = 128
    NCH = HALF // CH
    mesh = plsc.VectorSubcoreMesh(core_axis_name="c", subcore_axis_name="s",
                                  num_cores=NCORE, num_subcores=NSUB)

    @functools.partial(
        pl.kernel,
        mesh=mesh,
        out_type=(
            jax.ShapeDtypeStruct((B, R, D), jnp.float32),
            jax.ShapeDtypeStruct((B, R), jnp.float32),
        ),
        scratch_types=[
            pltpu.VMEM((R,), jnp.int32),
            pltpu.VMEM((NT,), jnp.float32),
            pltpu.VMEM((HALF,), jnp.float32),
            pltpu.VMEM((CH, D), jnp.float32),
            pltpu.VMEM((CH, D), jnp.float32),
            pltpu.SemaphoreType.DMA,
            pltpu.SemaphoreType.DMA,
            pltpu.SemaphoreType.DMA,
            pltpu.SemaphoreType.DMA,
        ],
        compiler_params=pltpu.CompilerParams(needs_layout_passes=False),
    )
    def gather_k(x_hbm, idx_hbm, sel_hbm, scale_hbm, idx_v, counts_v,
                 scale_v, buf0, buf1, gsem0, gsem1, osem0, osem1):
        wid = lax.axis_index("s") * NCORE + lax.axis_index("c")
        b = wid // WPB
        base = (wid % WPB) * HALF

        pltpu.sync_copy(idx_hbm.at[b], idx_v)

        bufs = (buf0, buf1)
        gsems = (gsem0, gsem1)
        osems = (osem0, osem1)

        def gath(g, p):
            return pltpu.async_copy(
                x_hbm.at[b].at[idx_v.at[pl.ds(base + g * CH, CH)]],
                bufs[p], gsems[p])

        gd = [gath(0, 0), None]
        od = [None, None]

        zeros16 = jnp.zeros((LANES,), jnp.float32)
        ones16 = jnp.ones((LANES,), jnp.float32)

        def zero_body(i, _):
            counts_v[pl.ds(i * LANES, LANES)] = zeros16
            return 0
        lax.fori_loop(0, NT // LANES, zero_body, 0, unroll=4)

        def hist_body(i, _):
            v = idx_v[pl.ds(i * LANES, LANES)]
            plsc.addupdate_scatter(counts_v, [v], ones16)
            return 0
        lax.fori_loop(0, R // LANES, hist_body, 0, unroll=4)

        def scale_body(i, _):
            iv = idx_v[pl.ds(base + i * LANES, LANES)]
            c = plsc.load_gather(counts_v, [iv])
            scale_v[pl.ds(i * LANES, LANES)] = 1.0 / c
            return 0
        lax.fori_loop(0, HALF // LANES, scale_body, 0, unroll=4)
        pltpu.sync_copy(scale_v, scale_hbm.at[b].at[pl.ds(base, HALF)])

        for g in range(NCH):
            p = g % 2
            gd[p].wait()
            if g + 1 < NCH:
                q = (g + 1) % 2
                if od[q] is not None:
                    od[q].wait()
                gd[q] = gath(g + 1, q)
            od[p] = pltpu.async_copy(
                bufs[p], sel_hbm.at[b].at[pl.ds(base + g * CH, CH)], osems[p])
        od[0].wait()
        od[1].wait()

    return gather_k


def _make_scatter(B, NT, D, R):
    PB = B // NCORE
    RT = NT // NSUB
    G = 64
    ZR = 64
    mesh = plsc.VectorSubcoreMesh(core_axis_name="c", subcore_axis_name="s",
                                  num_cores=NCORE, num_subcores=NSUB)

    @functools.partial(
        pl.kernel,
        mesh=mesh,
        out_type=jax.ShapeDtypeStruct((B, NT, D), jnp.float32),
        scratch_types=[
            pltpu.VMEM((RT, D), jnp.float32),
            pltpu.VMEM((3, G, D), jnp.float32),
            pltpu.VMEM((R,), jnp.int32),
            pltpu.VMEM((R + LANES,), jnp.int32),
            pltpu.VMEM((R + LANES,), jnp.int32),
            pltpu.SemaphoreType.DMA,
            pltpu.SemaphoreType.DMA,
            pltpu.SemaphoreType.DMA,
            pltpu.SemaphoreType.DMA,
            pltpu.SemaphoreType.DMA,
        ],
        compiler_params=pltpu.CompilerParams(needs_layout_passes=False),
    )
    def scatter_k(y_hbm, idx_hbm, out_hbm, acc_v, gbuf, idx_v,
                  mid_v, off_v, sem0, sem1, sem2, isem, osem):
        core = lax.axis_index("c")
        s = lax.axis_index("s")
        lo = s * RT
        zeros16 = jnp.zeros((LANES,), jnp.float32)
        zeros16i = jnp.zeros((LANES,), jnp.int32)
        sems = (sem0, sem1, sem2)

        def zm_body(i, _):
            mid_v[pl.ds(i * LANES, LANES)] = zeros16i
            return 0
        lax.fori_loop(0, (R + LANES) // LANES, zm_body, 0, unroll=4)

        def gstart(b, g, p, sp):
            pltpu.async_copy(
                y_hbm.at[b].at[mid_v.at[pl.ds(g * G, G)]], gbuf.at[p],
                sems[sp])

        def gwait(b, g, p, sp):
            pltpu.make_async_copy(
                y_hbm.at[b].at[mid_v.at[pl.ds(g * G, G)]], gbuf.at[p],
                sems[sp]).wait()

        pltpu.async_copy(idx_hbm.at[core * PB], idx_v, isem)

        def batch_body(i, _):
            b = core * PB + i

            with jax.named_scope("ph_idx"):
                pltpu.make_async_copy(idx_hbm.at[b], idx_v, isem).wait()

            def comp(j, n):
                v = idx_v[pl.ds(j * LANES, LANES)]
                m = (v >= lo) & (v < lo + RT)
                ids = lax.iota(jnp.int32, LANES) + j * LANES
                plsc.store_compressed(mid_v.at[pl.ds(n, LANES)], ids, mask=m)
                plsc.store_compressed(off_v.at[pl.ds(n, LANES)], v - lo, mask=m)
                return n + plsc.all_reduce_population_count(m)[0]
            with jax.named_scope("ph_comp"):
                n = lax.fori_loop(0, R // LANES, comp, 0, unroll=2)

            @pl.when(i + 1 < PB)
            def _():
                pltpu.async_copy(idx_hbm.at[b + 1], idx_v, isem)

            @pl.when(i > 0)
            def _():
                pltpu.make_async_copy(
                    acc_v, out_hbm.at[b].at[pl.ds(lo, RT)], osem).wait()

            def zacc(r, _):
                for c in range(D // LANES):
                    acc_v[r, pl.ds(c * LANES, LANES)] = zeros16
                return 0
            with jax.named_scope("ph_zacc"):
                lax.fori_loop(0, RT, zacc, 0, unroll=2)

            nch = (n + G - 1) // G

            for pp in range(3):
                @pl.when(nch > pp)
                def _(pp=pp):
                    gstart(b, pp, pp, pp)

            def chunk(g, _):
                p = g % 3

                with jax.named_scope("ph_gwait"):
                    for sp in range(3):
                        @pl.when(p == sp)
                        def _(sp=sp):
                            gwait(b, g, p, sp)

                nr = jnp.minimum(n - g * G, G)
                nfull = nr // LANES

                with jax.named_scope("ph_apply"):
                    def apply_full(q, _):
                        off16 = off_v[pl.ds(g * G + q * LANES, LANES)]
                        for j in range(LANES):
                            off = off16[j]
                            row = q * LANES + j
                            for c in range(D // LANES):
                                sl = pl.ds(c * LANES, LANES)
                                plsc.addupdate(acc_v.at[off, sl],
                                               gbuf[p, row, sl])
                        return 0
                    lax.fori_loop(0, nfull, apply_full, 0)

                    @pl.when(nfull * LANES < nr)
                    def _():
                        qq = nfull
                        off16 = off_v[pl.ds(g * G + qq * LANES, LANES)]
                        for j in range(LANES):
                            row = qq * LANES + j

                            @pl.when(row < nr)
                            def _():
                                off = off16[j]
                                for c in range(D // LANES):
                                    sl = pl.ds(c * LANES, LANES)
                                    plsc.addupdate(acc_v.at[off, sl],
                                                   gbuf[p, row, sl])

                for sp in range(3):
                    @pl.when((g + 3 < nch) & (p == sp))
                    def _(sp=sp):
                        gstart(b, g + 3, p, sp)
                return 0
            with jax.named_scope("ph_chunks"):
                lax.fori_loop(0, nch, chunk, 0)

            with jax.named_scope("ph_out"):
                pltpu.async_copy(acc_v, out_hbm.at[b].at[pl.ds(lo, RT)], osem)
            return 0

        lax.fori_loop(0, PB, batch_body, 0)
        pltpu.make_async_copy(
            acc_v, out_hbm.at[core * PB + PB - 1].at[pl.ds(lo, RT)],
            osem).wait()

    return scatter_k


def _make_mm(BK, NS, D, KB=8):
    def body(adj_ref, sel_ref, scale_ref, out_ref):
        for j in range(KB):
            a = adj_ref[j]
            v = sel_ref[j]
            sc = scale_ref[0, j, :]
            out_ref[j] = (
                jnp.dot(a, v, preferred_element_type=jnp.float32) * sc[:, None])

    return pl.pallas_call(
        body,
        grid=(BK // KB,),
        in_specs=[
            pl.BlockSpec((KB, NS, NS), lambda i: (i, 0, 0)),
            pl.BlockSpec((KB, NS, D), lambda i: (i, 0, 0)),
            pl.BlockSpec((1, KB, NS), lambda i: (i, 0, 0)),
        ],
        out_specs=pl.BlockSpec((KB, NS, D), lambda i: (i, 0, 0)),
        out_shape=jax.ShapeDtypeStruct((BK, NS, D), jnp.float32),
    )


def kernel(x, adj, batch_indices, indices):
    B, NT, D = x.shape
    _, K, NS, _ = adj.shape
    R = K * NS
    idx_flat = indices.reshape(B, R)

    KB = 16
    sel, scale = _make_gather(B, NT, D, R)(x, idx_flat)
    y = _make_mm(B * K, NS, D, KB)(
        adj.reshape(B * K, NS, NS),
        sel.reshape(B * K, NS, D),
        scale.reshape(B * K // KB, KB, NS),
    )
    return _make_scatter(B, NT, D, R)(y.reshape(B, R, D), idx_flat)


# speedup vs baseline: 1.3644x; 1.0149x over previous
import functools

import jax
import jax.numpy as jnp
from jax import lax
from jax.experimental import pallas as pl
from jax.experimental.pallas import tpu as pltpu
from jax.experimental.pallas import tpu_sc as plsc

NCORE, NSUB, LANES = 2, 16, 16
NW = NCORE * NSUB


def _make_gather(B, NT, D, R):
    WPB = NW // B
    HALF = R // WPB
    CH = 128
    NCH = HALF // CH
    mesh = plsc.VectorSubcoreMesh(core_axis_name="c", subcore_axis_name="s",
                                  num_cores=NCORE, num_subcores=NSUB)

    @functools.partial(
        pl.kernel,
        mesh=mesh,
        out_type=(
            jax.ShapeDtypeStruct((B, R, D), jnp.float32),
            jax.ShapeDtypeStruct((B, R), jnp.float32),
        ),
        scratch_types=[
            pltpu.VMEM((R,), jnp.int32),
            pltpu.VMEM((NT,), jnp.float32),
            pltpu.VMEM((HALF,), jnp.float32),
            pltpu.VMEM((CH, D), jnp.float32),
            pltpu.VMEM((CH, D), jnp.float32),
            pltpu.SemaphoreType.DMA,
            pltpu.SemaphoreType.DMA,
            pltpu.SemaphoreType.DMA,
            pltpu.SemaphoreType.DMA,
        ],
        compiler_params=pltpu.CompilerParams(needs_layout_passes=False),
    )
    def gather_k(x_hbm, idx_hbm, sel_hbm, scale_hbm, idx_v, counts_v,
                 scale_v, buf0, buf1, gsem0, gsem1, osem0, osem1):
        wid = lax.axis_index("s") * NCORE + lax.axis_index("c")
        b = wid // WPB
        base = (wid % WPB) * HALF

        pltpu.sync_copy(idx_hbm.at[b], idx_v)

        bufs = (buf0, buf1)
        gsems = (gsem0, gsem1)
        osems = (osem0, osem1)

        def gath(g, p):
            return pltpu.async_copy(
                x_hbm.at[b].at[idx_v.at[pl.ds(base + g * CH, CH)]],
                bufs[p], gsems[p])

        gd = [gath(0, 0), None]
        od = [None, None]

        zeros16 = jnp.zeros((LANES,), jnp.float32)
        ones16 = jnp.ones((LANES,), jnp.float32)

        def zero_body(i, _):
            counts_v[pl.ds(i * LANES, LANES)] = zeros16
            return 0
        lax.fori_loop(0, NT // LANES, zero_body, 0, unroll=4)

        def hist_body(i, _):
            v = idx_v[pl.ds(i * LANES, LANES)]
            plsc.addupdate_scatter(counts_v, [v], ones16)
            return 0
        lax.fori_loop(0, R // LANES, hist_body, 0, unroll=4)

        def scale_body(i, _):
            iv = idx_v[pl.ds(base + i * LANES, LANES)]
            c = plsc.load_gather(counts_v, [iv])
            scale_v[pl.ds(i * LANES, LANES)] = 1.0 / c
            return 0
        lax.fori_loop(0, HALF // LANES, scale_body, 0, unroll=4)
        pltpu.sync_copy(scale_v, scale_hbm.at[b].at[pl.ds(base, HALF)])

        for g in range(NCH):
            p = g % 2
            gd[p].wait()
            if g + 1 < NCH:
                q = (g + 1) % 2
                if od[q] is not None:
                    od[q].wait()
                gd[q] = gath(g + 1, q)
            od[p] = pltpu.async_copy(
                bufs[p], sel_hbm.at[b].at[pl.ds(base + g * CH, CH)], osems[p])
        od[0].wait()
        od[1].wait()

    return gather_k


def _make_scatter(B, NT, D, R):
    PB = B // NCORE
    RT = NT // NSUB
    G = 64
    ZR = 64
    mesh = plsc.VectorSubcoreMesh(core_axis_name="c", subcore_axis_name="s",
                                  num_cores=NCORE, num_subcores=NSUB)

    @functools.partial(
        pl.kernel,
        mesh=mesh,
        out_type=jax.ShapeDtypeStruct((B, NT, D), jnp.float32),
        scratch_types=[
            pltpu.VMEM((RT, D), jnp.float32),
            pltpu.VMEM((3, G, D), jnp.float32),
            pltpu.VMEM((R,), jnp.int32),
            pltpu.VMEM((R + LANES,), jnp.int32),
            pltpu.VMEM((R + LANES,), jnp.int32),
            pltpu.SemaphoreType.DMA,
            pltpu.SemaphoreType.DMA,
            pltpu.SemaphoreType.DMA,
            pltpu.SemaphoreType.DMA,
            pltpu.SemaphoreType.DMA,
        ],
        compiler_params=pltpu.CompilerParams(needs_layout_passes=False),
    )
    def scatter_k(y_hbm, idx_hbm, out_hbm, acc_v, gbuf, idx_v,
                  mid_v, off_v, sem0, sem1, sem2, isem, osem):
        core = lax.axis_index("c")
        s = lax.axis_index("s")
        lo = s * RT
        zeros16 = jnp.zeros((LANES,), jnp.float32)
        zeros16i = jnp.zeros((LANES,), jnp.int32)
        sems = (sem0, sem1, sem2)

        def zm_body(i, _):
            mid_v[pl.ds(i * LANES, LANES)] = zeros16i
            return 0
        lax.fori_loop(0, (R + LANES) // LANES, zm_body, 0, unroll=4)

        def gstart(b, g, p, sp):
            pltpu.async_copy(
                y_hbm.at[b].at[mid_v.at[pl.ds(g * G, G)]], gbuf.at[p],
                sems[sp])

        def gwait(b, g, p, sp):
            pltpu.make_async_copy(
                y_hbm.at[b].at[mid_v.at[pl.ds(g * G, G)]], gbuf.at[p],
                sems[sp]).wait()

        pltpu.async_copy(idx_hbm.at[core * PB], idx_v, isem)

        def batch_body(i, _):
            b = core * PB + i

            with jax.named_scope("ph_idx"):
                pltpu.make_async_copy(idx_hbm.at[b], idx_v, isem).wait()

            def comp(j, n):
                v = idx_v[pl.ds(j * LANES, LANES)]
                m = (v >= lo) & (v < lo + RT)
                ids = lax.iota(jnp.int32, LANES) + j * LANES
                plsc.store_compressed(mid_v.at[pl.ds(n, LANES)], ids, mask=m)
                plsc.store_compressed(off_v.at[pl.ds(n, LANES)], v - lo, mask=m)
                return n + plsc.all_reduce_population_count(m)[0]
            with jax.named_scope("ph_comp"):
                n = lax.fori_loop(0, R // LANES, comp, 0, unroll=2)

            @pl.when(i + 1 < PB)
            def _():
                pltpu.async_copy(idx_hbm.at[b + 1], idx_v, isem)

            @pl.when(i > 0)
            def _():
                pltpu.make_async_copy(
                    acc_v, out_hbm.at[b].at[pl.ds(lo, RT)], osem).wait()

            def zacc(r, _):
                for c in range(D // LANES):
                    acc_v[r, pl.ds(c * LANES, LANES)] = zeros16
                return 0
            with jax.named_scope("ph_zacc"):
                lax.fori_loop(0, RT, zacc, 0, unroll=2)

            nch = (n + G - 1) // G

            for pp in range(3):
                @pl.when(nch > pp)
                def _(pp=pp):
                    gstart(b, pp, pp, pp)

            def chunk(g, _):
                p = g % 3

                with jax.named_scope("ph_gwait"):
                    for sp in range(3):
                        @pl.when(p == sp)
                        def _(sp=sp):
                            gwait(b, g, p, sp)

                nr = jnp.minimum(n - g * G, G)
                nfull = nr // LANES

                with jax.named_scope("ph_apply"):
                    def apply_full(q, _):
                        off16 = off_v[pl.ds(g * G + q * LANES, LANES)]
                        for j in range(LANES):
                            off = off16[j]
                            row = q * LANES + j
                            for c in range(D // LANES):
                                sl = pl.ds(c * LANES, LANES)
                                plsc.addupdate(acc_v.at[off, sl],
                                               gbuf[p, row, sl])
                        return 0
                    lax.fori_loop(0, nfull, apply_full, 0)

                    @pl.when(nfull * LANES < nr)
                    def _():
                        qq = nfull
                        off16 = off_v[pl.ds(g * G + qq * LANES, LANES)]
                        for j in range(LANES):
                            row = qq * LANES + j

                            @pl.when(row < nr)
                            def _():
                                off = off16[j]
                                for c in range(D // LANES):
                                    sl = pl.ds(c * LANES, LANES)
                                    plsc.addupdate(acc_v.at[off, sl],
                                                   gbuf[p, row, sl])

                for sp in range(3):
                    @pl.when((g + 3 < nch) & (p == sp))
                    def _(sp=sp):
                        gstart(b, g + 3, p, sp)
                return 0
            with jax.named_scope("ph_chunks"):
                lax.fori_loop(0, nch, chunk, 0)

            with jax.named_scope("ph_out"):
                pltpu.async_copy(acc_v, out_hbm.at[b].at[pl.ds(lo, RT)], osem)
            return 0

        lax.fori_loop(0, PB, batch_body, 0)
        pltpu.make_async_copy(
            acc_v, out_hbm.at[core * PB + PB - 1].at[pl.ds(lo, RT)],
            osem).wait()

    return scatter_k


def _make_mm(BK, NS, D, KB=8):
    def body(adj_ref, sel_ref, scale_ref, out_ref):
        for j in range(KB):
            a = adj_ref[j]
            v = sel_ref[j]
            sc = scale_ref[0, j, :]
            out_ref[j] = (
                jnp.dot(a, v, preferred_element_type=jnp.float32) * sc[:, None])

    return pl.pallas_call(
        body,
        grid=(BK // KB,),
        in_specs=[
            pl.BlockSpec((KB, NS, NS), lambda i: (i, 0, 0)),
            pl.BlockSpec((KB, NS, D), lambda i: (i, 0, 0)),
            pl.BlockSpec((1, KB, NS), lambda i: (i, 0, 0)),
        ],
        out_specs=pl.BlockSpec((KB, NS, D), lambda i: (i, 0, 0)),
        out_shape=jax.ShapeDtypeStruct((BK, NS, D), jnp.float32),
    )


def kernel(x, adj, batch_indices, indices):
    B, NT, D = x.shape
    _, K, NS, _ = adj.shape
    R = K * NS
    idx_flat = indices.reshape(B, R)

    KB = 32
    sel, scale = _make_gather(B, NT, D, R)(x, idx_flat)
    y = _make_mm(B * K, NS, D, KB)(
        adj.reshape(B * K, NS, NS),
        sel.reshape(B * K, NS, D),
        scale.reshape(B * K // KB, KB, NS),
    )
    return _make_scatter(B, NT, D, R)(y.reshape(B, R, D), idx_flat)


# speedup vs baseline: 1.3663x; 1.0014x over previous
import functools

import jax
import jax.numpy as jnp
from jax import lax
from jax.experimental import pallas as pl
from jax.experimental.pallas import tpu as pltpu
from jax.experimental.pallas import tpu_sc as plsc

NCORE, NSUB, LANES = 2, 16, 16
NW = NCORE * NSUB


def _make_gather(B, NT, D, R):
    WPB = NW // B
    HALF = R // WPB
    CH = 128
    NCH = HALF // CH
    mesh = plsc.VectorSubcoreMesh(core_axis_name="c", subcore_axis_name="s",
                                  num_cores=NCORE, num_subcores=NSUB)

    @functools.partial(
        pl.kernel,
        mesh=mesh,
        out_type=(
            jax.ShapeDtypeStruct((B, R, D), jnp.float32),
            jax.ShapeDtypeStruct((B, R), jnp.float32),
        ),
        scratch_types=[
            pltpu.VMEM((R,), jnp.int32),
            pltpu.VMEM((NT,), jnp.float32),
            pltpu.VMEM((HALF,), jnp.float32),
            pltpu.VMEM((CH, D), jnp.float32),
            pltpu.VMEM((CH, D), jnp.float32),
            pltpu.SemaphoreType.DMA,
            pltpu.SemaphoreType.DMA,
            pltpu.SemaphoreType.DMA,
            pltpu.SemaphoreType.DMA,
        ],
        compiler_params=pltpu.CompilerParams(needs_layout_passes=False),
    )
    def gather_k(x_hbm, idx_hbm, sel_hbm, scale_hbm, idx_v, counts_v,
                 scale_v, buf0, buf1, gsem0, gsem1, osem0, osem1):
        wid = lax.axis_index("s") * NCORE + lax.axis_index("c")
        b = wid // WPB
        base = (wid % WPB) * HALF

        pltpu.sync_copy(idx_hbm.at[b], idx_v)

        bufs = (buf0, buf1)
        gsems = (gsem0, gsem1)
        osems = (osem0, osem1)

        def gath(g, p):
            return pltpu.async_copy(
                x_hbm.at[b].at[idx_v.at[pl.ds(base + g * CH, CH)]],
                bufs[p], gsems[p])

        gd = [gath(0, 0), None]
        od = [None, None]

        zeros16 = jnp.zeros((LANES,), jnp.float32)
        ones16 = jnp.ones((LANES,), jnp.float32)

        def zero_body(i, _):
            counts_v[pl.ds(i * LANES, LANES)] = zeros16
            return 0
        lax.fori_loop(0, NT // LANES, zero_body, 0, unroll=4)

        def hist_body(i, _):
            v = idx_v[pl.ds(i * LANES, LANES)]
            plsc.addupdate_scatter(counts_v, [v], ones16)
            return 0
        lax.fori_loop(0, R // LANES, hist_body, 0, unroll=4)

        def scale_body(i, _):
            iv = idx_v[pl.ds(base + i * LANES, LANES)]
            c = plsc.load_gather(counts_v, [iv])
            scale_v[pl.ds(i * LANES, LANES)] = 1.0 / c
            return 0
        lax.fori_loop(0, HALF // LANES, scale_body, 0, unroll=4)
        pltpu.sync_copy(scale_v, scale_hbm.at[b].at[pl.ds(base, HALF)])

        for g in range(NCH):
            p = g % 2
            gd[p].wait()
            if g + 1 < NCH:
                q = (g + 1) % 2
                if od[q] is not None:
                    od[q].wait()
                gd[q] = gath(g + 1, q)
            od[p] = pltpu.async_copy(
                bufs[p], sel_hbm.at[b].at[pl.ds(base + g * CH, CH)], osems[p])
        od[0].wait()
        od[1].wait()

    return gather_k


def _make_scatter(B, NT, D, R):
    PB = B // NCORE
    RT = NT // NSUB
    G = 64
    ZR = 64
    mesh = plsc.VectorSubcoreMesh(core_axis_name="c", subcore_axis_name="s",
                                  num_cores=NCORE, num_subcores=NSUB)

    @functools.partial(
        pl.kernel,
        mesh=mesh,
        out_type=jax.ShapeDtypeStruct((B, NT, D), jnp.float32),
        scratch_types=[
            pltpu.VMEM((RT, D), jnp.float32),
            pltpu.VMEM((3, G, D), jnp.float32),
            pltpu.VMEM((R,), jnp.int32),
            pltpu.VMEM((R + LANES,), jnp.int32),
            pltpu.VMEM((R + LANES,), jnp.int32),
            pltpu.SemaphoreType.DMA,
            pltpu.SemaphoreType.DMA,
            pltpu.SemaphoreType.DMA,
            pltpu.SemaphoreType.DMA,
            pltpu.SemaphoreType.DMA,
        ],
        compiler_params=pltpu.CompilerParams(needs_layout_passes=False),
    )
    def scatter_k(y_hbm, idx_hbm, out_hbm, acc_v, gbuf, idx_v,
                  mid_v, off_v, sem0, sem1, sem2, isem, osem):
        core = lax.axis_index("c")
        s = lax.axis_index("s")
        lo = s * RT
        zeros16 = jnp.zeros((LANES,), jnp.float32)
        zeros16i = jnp.zeros((LANES,), jnp.int32)
        sems = (sem0, sem1, sem2)

        def zm_body(i, _):
            mid_v[pl.ds(i * LANES, LANES)] = zeros16i
            return 0
        lax.fori_loop(0, (R + LANES) // LANES, zm_body, 0, unroll=4)

        def gstart(b, g, p, sp):
            pltpu.async_copy(
                y_hbm.at[b].at[mid_v.at[pl.ds(g * G, G)]], gbuf.at[p],
                sems[sp])

        def gwait(b, g, p, sp):
            pltpu.make_async_copy(
                y_hbm.at[b].at[mid_v.at[pl.ds(g * G, G)]], gbuf.at[p],
                sems[sp]).wait()

        pltpu.async_copy(idx_hbm.at[core * PB], idx_v, isem)

        def batch_body(i, _):
            b = core * PB + i

            with jax.named_scope("ph_idx"):
                pltpu.make_async_copy(idx_hbm.at[b], idx_v, isem).wait()

            def comp(j, n):
                v = idx_v[pl.ds(j * LANES, LANES)]
                m = (v >= lo) & (v < lo + RT)
                ids = lax.iota(jnp.int32, LANES) + j * LANES
                plsc.store_compressed(mid_v.at[pl.ds(n, LANES)], ids, mask=m)
                plsc.store_compressed(off_v.at[pl.ds(n, LANES)], v - lo, mask=m)
                return n + plsc.all_reduce_population_count(m)[0]
            with jax.named_scope("ph_comp"):
                n = lax.fori_loop(0, R // LANES, comp, 0, unroll=2)

            @pl.when(i + 1 < PB)
            def _():
                pltpu.async_copy(idx_hbm.at[b + 1], idx_v, isem)

            @pl.when(i > 0)
            def _():
                pltpu.make_async_copy(
                    acc_v, out_hbm.at[b].at[pl.ds(lo, RT)], osem).wait()

            def zacc(r, _):
                for c in range(D // LANES):
                    acc_v[r, pl.ds(c * LANES, LANES)] = zeros16
                return 0
            with jax.named_scope("ph_zacc"):
                lax.fori_loop(0, RT, zacc, 0, unroll=2)

            nch = (n + G - 1) // G

            for pp in range(3):
                @pl.when(nch > pp)
                def _(pp=pp):
                    gstart(b, pp, pp, pp)

            def chunk(g, _):
                p = g % 3

                with jax.named_scope("ph_gwait"):
                    for sp in range(3):
                        @pl.when(p == sp)
                        def _(sp=sp):
                            gwait(b, g, p, sp)

                nr = jnp.minimum(n - g * G, G)
                nfull = nr // LANES

                with jax.named_scope("ph_apply"):
                    def apply_full(q, _):
                        off16 = off_v[pl.ds(g * G + q * LANES, LANES)]
                        for j in range(LANES):
                            off = off16[j]
                            row = q * LANES + j
                            for c in range(D // LANES):
                                sl = pl.ds(c * LANES, LANES)
                                plsc.addupdate(acc_v.at[off, sl],
                                               gbuf[p, row, sl])
                        return 0
                    lax.fori_loop(0, nfull, apply_full, 0)

                    @pl.when(nfull * LANES < nr)
                    def _():
                        qq = nfull
                        off16 = off_v[pl.ds(g * G + qq * LANES, LANES)]
                        for j in range(LANES):
                            row = qq * LANES + j

                            @pl.when(row < nr)
                            def _():
                                off = off16[j]
                                for c in range(D // LANES):
                                    sl = pl.ds(c * LANES, LANES)
                                    plsc.addupdate(acc_v.at[off, sl],
                                                   gbuf[p, row, sl])

                for sp in range(3):
                    @pl.when((g + 3 < nch) & (p == sp))
                    def _(sp=sp):
                        gstart(b, g + 3, p, sp)
                return 0
            with jax.named_scope("ph_chunks"):
                lax.fori_loop(0, nch, chunk, 0)

            with jax.named_scope("ph_out"):
                pltpu.async_copy(acc_v, out_hbm.at[b].at[pl.ds(lo, RT)], osem)
            return 0

        lax.fori_loop(0, PB, batch_body, 0)
        pltpu.make_async_copy(
            acc_v, out_hbm.at[core * PB + PB - 1].at[pl.ds(lo, RT)],
            osem).wait()

    return scatter_k


def _make_mm(BK, NS, D, KB=8):
    def body(adj_ref, sel_ref, scale_ref, out_ref):
        for j in range(KB):
            a = adj_ref[j]
            v = sel_ref[j]
            sc = scale_ref[0, j, :]
            out_ref[j] = (
                jnp.dot(a, v, preferred_element_type=jnp.float32) * sc[:, None])

    return pl.pallas_call(
        body,
        grid=(BK // KB,),
        in_specs=[
            pl.BlockSpec((KB, NS, NS), lambda i: (i, 0, 0)),
            pl.BlockSpec((KB, NS, D), lambda i: (i, 0, 0)),
            pl.BlockSpec((1, KB, NS), lambda i: (i, 0, 0)),
        ],
        out_specs=pl.BlockSpec((KB, NS, D), lambda i: (i, 0, 0)),
        out_shape=jax.ShapeDtypeStruct((BK, NS, D), jnp.float32),
    )


def kernel(x, adj, batch_indices, indices):
    B, NT, D = x.shape
    _, K, NS, _ = adj.shape
    R = K * NS
    idx_flat = indices.reshape(B, R)

    KB = 64
    sel, scale = _make_gather(B, NT, D, R)(x, idx_flat)
    y = _make_mm(B * K, NS, D, KB)(
        adj.reshape(B * K, NS, NS),
        sel.reshape(B * K, NS, D),
        scale.reshape(B * K // KB, KB, NS),
    )
    return _make_scatter(B, NT, D, R)(y.reshape(B, R, D), idx_flat)
